# Initial kernel scaffold; baseline (speedup 1.0000x reference)
import jax
import jax.numpy as jnp
from jax.experimental import pallas as pl


def kernel(token_ids, weights):
    raise NotImplementedError("write your pallas kernel here")


# speedup vs baseline: 1.8784x; 1.8784x over previous
import functools

import jax
import jax.numpy as jnp
from jax import lax
from jax.experimental import pallas as pl
from jax.experimental.pallas import tpu as pltpu
from jax.experimental.pallas import tpu_sc as plsc

NC, NS = 2, 16
NW = NC * NS
BATCH, SEQ, D = 16384, 50, 64
TOT = BATCH * SEQ
PER_W = TOT // NW
CHUNK = 256
KS = CHUNK // 128
NCH = PER_W // CHUNK
NBUF = 4
GLEAD = 2
ILEAD = 3
IDX_ROWS = TOT // 128
W_ROWS128 = PER_W // 128

_mesh = plsc.VectorSubcoreMesh(core_axis_name="c", subcore_axis_name="s")


@functools.partial(
    pl.kernel,
    out_type=jax.ShapeDtypeStruct((TOT, D), jnp.float32),
    mesh=_mesh,
    scratch_types=[
        pltpu.VMEM((NBUF, KS, 128), jnp.int32),
        pltpu.VMEM((NBUF, CHUNK, D), jnp.float32),
        pltpu.SemaphoreType.DMA((NBUF,)),
        pltpu.SemaphoreType.DMA((NBUF,)),
        pltpu.SemaphoreType.DMA((NBUF,)),
    ],
    compiler_params=pltpu.CompilerParams(use_tc_tiling_on_sc=False),
)
def _embed(tok_hbm, w_hbm, out_hbm, idx_v, rows_v, isem, gsem, osem):
    wid = lax.axis_index("s") * NC + lax.axis_index("c")
    row0 = wid * W_ROWS128
    out0 = wid * PER_W

    def start_idx(slot, g):
        pltpu.async_copy(tok_hbm.at[pl.ds(row0 + g * KS, KS)],
                         idx_v.at[slot], isem.at[slot])

    def wait_idx(slot):
        pltpu.make_async_copy(tok_hbm.at[pl.ds(0, KS)],
                              idx_v.at[slot], isem.at[slot]).wait()

    def start_gather(slot):
        for j in range(KS):
            pltpu.async_copy(w_hbm.at[idx_v.at[slot, j]],
                             rows_v.at[slot, pl.ds(j * 128, 128)],
                             gsem.at[slot])

    def wait_gather(slot):
        for j in range(KS):
            pltpu.make_async_copy(w_hbm.at[idx_v.at[slot, j]],
                                  rows_v.at[slot, pl.ds(j * 128, 128)],
                                  gsem.at[slot]).wait()

    def start_out(slot, g):
        pltpu.async_copy(rows_v.at[slot],
                         out_hbm.at[pl.ds(out0 + g * CHUNK, CHUNK)],
                         osem.at[slot])

    def wait_out(slot):
        pltpu.make_async_copy(rows_v.at[slot],
                              out_hbm.at[pl.ds(0, CHUNK)], osem.at[slot]).wait()

    def step(i, b, do_wait_out, do_gather, do_idx):
        wait_gather(b)
        start_out(b, i)
        if do_gather:
            bg = (b + GLEAD) % NBUF
            if do_wait_out:
                wait_out(bg)
            wait_idx(bg)
            start_gather(bg)
        if do_idx:
            start_idx((b + ILEAD) % NBUF, i + ILEAD)

    for g in range(ILEAD):
        start_idx(g, g)
    for g in range(GLEAD):
        wait_idx(g)
        start_gather(g)

    for b in range(NBUF):
        step(b, b, do_wait_out=(b >= GLEAD), do_gather=True, do_idx=True)

    def outer_body(o):
        i0 = o * NBUF
        for b in range(NBUF):
            step(i0 + b, b, do_wait_out=True, do_gather=True, do_idx=True)
    pl.loop(1, NCH // NBUF - 1)(outer_body)

    iL = NCH - NBUF
    for b in range(NBUF):
        i = iL + b
        step(i, b,
             do_wait_out=(i + GLEAD < NCH),
             do_gather=(i + GLEAD < NCH),
             do_idx=(i + ILEAD < NCH))

    for b in range(NBUF):
        wait_out(b)


def kernel(token_ids, weights):
    tok = token_ids.reshape(IDX_ROWS, 128).astype(jnp.int32)
    out = _embed(tok, weights)
    return out.reshape(BATCH, SEQ, D)
